# Initial kernel scaffold; baseline (speedup 1.0000x reference)
import jax
import jax.numpy as jnp
from jax.experimental import pallas as pl


def kernel(inputs, Wr, br, We, be):
    raise NotImplementedError("write your pallas kernel here")


# speedup vs baseline: 1.4625x; 1.4625x over previous
import jax
import jax.numpy as jnp
from jax.experimental import pallas as pl
from jax.experimental.pallas import tpu as pltpu

TOKEN_BLOCK = 512


def _dense_body(x_ref, wr_ref, br_ref, we_ref, be_ref, o_ref, probs_ref):
    e = pl.program_id(1)

    @pl.when(e == 0)
    def _():
        logits = (
            jnp.dot(x_ref[...], wr_ref[...], preferred_element_type=jnp.float32)
            + br_ref[...]
        )
        m = jnp.max(logits, axis=-1, keepdims=True)
        ex = jnp.exp(logits - m)
        probs_ref[...] = ex / jnp.sum(ex, axis=-1, keepdims=True)

    probs = probs_ref[...]
    ne = probs.shape[-1]
    lane = jax.lax.broadcasted_iota(jnp.int32, probs.shape, 1)
    sel = (lane == e).astype(jnp.float32)
    p_e = jnp.sum(probs * sel, axis=-1, keepdims=True)
    gt = jnp.sum((probs > p_e).astype(jnp.float32), axis=-1, keepdims=True)
    eqb = jnp.sum(
        ((probs == p_e) & (lane < e)).astype(jnp.float32), axis=-1, keepdims=True
    )
    coef = jnp.where(gt + eqb < 2.0, p_e, 0.0)

    y = jnp.dot(x_ref[...], we_ref[0], preferred_element_type=jnp.float32)
    y = jnp.maximum(y + be_ref[0], 0.0) * coef

    @pl.when(e == 0)
    def _():
        o_ref[...] = y

    @pl.when(e != 0)
    def _():
        o_ref[...] += y


def kernel(inputs, Wr, br, We, be):
    n, d = inputs.shape
    ne = Wr.shape[1]
    u = We.shape[2]
    tb = TOKEN_BLOCK
    out = pl.pallas_call(
        _dense_body,
        grid=(n // tb, ne),
        in_specs=[
            pl.BlockSpec((tb, d), lambda t, e: (t, 0)),
            pl.BlockSpec((d, ne), lambda t, e: (0, 0)),
            pl.BlockSpec((1, ne), lambda t, e: (0, 0)),
            pl.BlockSpec((1, d, u), lambda t, e: (e, 0, 0)),
            pl.BlockSpec((1, 1, u), lambda t, e: (e, 0, 0)),
        ],
        out_specs=pl.BlockSpec((tb, u), lambda t, e: (t, 0)),
        out_shape=jax.ShapeDtypeStruct((n, u), jnp.float32),
        scratch_shapes=[pltpu.VMEM((tb, ne), jnp.float32)],
        compiler_params=pltpu.CompilerParams(
            dimension_semantics=("parallel", "arbitrary")
        ),
    )(inputs, Wr, br.reshape(1, ne), We, be.reshape(ne, 1, u))
    return out


# speedup vs baseline: 1.6500x; 1.1282x over previous
import functools

import jax
import jax.numpy as jnp
from jax import lax
from jax.experimental import pallas as pl
from jax.experimental.pallas import tpu as pltpu
from jax.experimental.pallas import tpu_sc as plsc

NE = 16
NT = 4096
NA = 2 * NT
BLK = 256
NBMAX = NA // BLK + NE
CAP = NBMAX * BLK
CHUNK = 128
NCHUNK = NA // CHUNK
NC, NS = 2, 16
NW = NC * NS
TPW = NT // NW


def _router_body(x_ref, wr_ref, br_ref, probs_ref):
    logits = (
        jnp.dot(x_ref[...], wr_ref[...], preferred_element_type=jnp.float32)
        + br_ref[...]
    )
    m = jnp.max(logits, axis=-1, keepdims=True)
    ex = jnp.exp(logits - m)
    probs_ref[...] = ex / jnp.sum(ex, axis=-1, keepdims=True)


def _router(x, Wr, br):
    n, d = x.shape
    tb = 1024
    return pl.pallas_call(
        _router_body,
        grid=(n // tb,),
        in_specs=[
            pl.BlockSpec((tb, d), lambda t: (t, 0)),
            pl.BlockSpec((d, NE), lambda t: (0, 0)),
            pl.BlockSpec((1, NE), lambda t: (0, 0)),
        ],
        out_specs=pl.BlockSpec((tb, NE), lambda t: (t, 0)),
        out_shape=jax.ShapeDtypeStruct((n, NE), jnp.float32),
    )(x, Wr, br.reshape(1, NE))


def _routing_body(probs_ref, slots_ref, w_ref, bexp_ref, oh_ref, call_ref, p_ref):
    probs = probs_ref[...]
    lane = lax.broadcasted_iota(jnp.int32, probs.shape, 1)
    m1 = jnp.max(probs, axis=-1, keepdims=True)
    a1 = jnp.min(jnp.where(probs == m1, lane, NE), axis=-1, keepdims=True)
    rest = jnp.where(lane == a1, -jnp.inf, probs)
    m2 = jnp.max(rest, axis=-1, keepdims=True)
    a2 = jnp.min(jnp.where(rest == m2, lane, NE), axis=-1, keepdims=True)
    w_ref[:, 0:1] = m1
    w_ref[:, 1:2] = m2
    oh_ref[0:NT] = (lane == a1).astype(jnp.float32)
    oh_ref[NT:NA] = (lane == a2).astype(jnp.float32)

    r_i = lax.broadcasted_iota(jnp.int32, (CHUNK, CHUNK), 0)
    c_i = lax.broadcasted_iota(jnp.int32, (CHUNK, CHUNK), 1)
    tri = (r_i >= c_i).astype(jnp.float32)

    def chunk_cumsum(c, _):
        oc = oh_ref[pl.ds(c * CHUNK, CHUNK), :]
        cc = jnp.dot(tri, oc, preferred_element_type=jnp.float32)
        call_ref[pl.ds(c * CHUNK, CHUNK), :] = cc
        p_ref[pl.ds(c, 1), :] = cc[CHUNK - 1 : CHUNK, :]
        return 0

    lax.fori_loop(0, NCHUNK, chunk_cumsum, 0)

    r64 = lax.broadcasted_iota(jnp.int32, (NCHUNK, NCHUNK), 0)
    c64 = lax.broadcasted_iota(jnp.int32, (NCHUNK, NCHUNK), 1)
    tri64 = (r64 > c64).astype(jnp.float32)
    totals = p_ref[...]
    pref = jnp.dot(tri64, totals, preferred_element_type=jnp.float32)
    counts = pref[NCHUNK - 1 : NCHUNK, :] + totals[NCHUNK - 1 : NCHUNK, :]

    nb = jnp.floor((counts + (BLK - 1)) * (1.0 / BLK))
    le = lax.broadcasted_iota(jnp.int32, (NE, NE), 0)
    lf = lax.broadcasted_iota(jnp.int32, (NE, NE), 1)
    u_excl = (le < lf).astype(jnp.float32)
    u_incl = (le <= lf).astype(jnp.float32)
    start_rows = BLK * jnp.dot(nb, u_excl, preferred_element_type=jnp.float32)
    cum_incl = jnp.dot(nb, u_incl, preferred_element_type=jnp.float32)
    total_blocks = cum_incl[0:1, NE - 1 : NE]
    bi = lax.broadcasted_iota(jnp.int32, (NBMAX, NE), 0).astype(jnp.float32)
    bexp = jnp.sum((cum_incl <= bi).astype(jnp.float32), axis=-1, keepdims=True)
    act = bi[:, 0:1] < total_blocks
    bexp_ref[...] = jnp.where(act, bexp, NE - 1.0).astype(jnp.int32)

    p_ref[...] = pref

    def chunk_slot(c, _):
        oc = oh_ref[pl.ds(c * CHUNK, CHUNK), :]
        r_incl = call_ref[pl.ds(c * CHUNK, CHUNK), :] + p_ref[pl.ds(c, 1), :]
        r_excl = r_incl - oc
        slot = jnp.sum((start_rows + r_excl) * oc, axis=-1, keepdims=True)
        slots_ref[pl.ds(c * CHUNK, CHUNK), :] = slot.astype(jnp.int32)
        return 0

    lax.fori_loop(0, NCHUNK, chunk_slot, 0)


def _routing(probs):
    return pl.pallas_call(
        _routing_body,
        in_specs=[pl.BlockSpec((NT, NE), lambda: (0, 0))],
        out_specs=[
            pl.BlockSpec((NA, 1), lambda: (0, 0)),
            pl.BlockSpec((NT, 2), lambda: (0, 0)),
            pl.BlockSpec((NBMAX, 1), lambda: (0, 0)),
        ],
        out_shape=[
            jax.ShapeDtypeStruct((NA, 1), jnp.int32),
            jax.ShapeDtypeStruct((NT, 2), jnp.float32),
            jax.ShapeDtypeStruct((NBMAX, 1), jnp.int32),
        ],
        scratch_shapes=[
            pltpu.VMEM((NA, NE), jnp.float32),
            pltpu.VMEM((NA, NE), jnp.float32),
            pltpu.VMEM((NCHUNK, NE), jnp.float32),
        ],
    )(probs)


def _dispatch_body(x_hbm, s0_hbm, s1_hbm, xs_hbm, rows_v, i0_v, i1_v, sem):
    wid = lax.axis_index("s") * NC + lax.axis_index("c")
    rows_per = 32

    def body(ch, _):
        base = wid * TPW + ch * rows_per
        pltpu.sync_copy(x_hbm.at[pl.ds(base, rows_per)], rows_v)
        pltpu.sync_copy(s0_hbm.at[pl.ds(base, rows_per)], i0_v)
        pltpu.sync_copy(s1_hbm.at[pl.ds(base, rows_per)], i1_v)
        pltpu.async_copy(rows_v, xs_hbm.at[i0_v], sem).wait()
        pltpu.async_copy(rows_v, xs_hbm.at[i1_v], sem).wait()
        return 0

    lax.fori_loop(0, TPW // rows_per, body, 0)


def _dispatch(x, s0, s1):
    d = x.shape[1]
    mesh = plsc.VectorSubcoreMesh(
        core_axis_name="c", subcore_axis_name="s", num_cores=NC, num_subcores=NS
    )
    f = functools.partial(
        pl.kernel,
        out_type=jax.ShapeDtypeStruct((CAP, d), jnp.float32),
        mesh=mesh,
        scratch_types=[
            pltpu.VMEM((32, d), jnp.float32),
            pltpu.VMEM((32,), jnp.int32),
            pltpu.VMEM((32,), jnp.int32),
            pltpu.SemaphoreType.DMA,
        ],
    )(_dispatch_body)
    return f(x, s0, s1)


def _gmm_body(bexp_smem, x_ref, we_ref, be_ref, y_ref):
    del bexp_smem
    y = jnp.dot(x_ref[...], we_ref[0], preferred_element_type=jnp.float32)
    y_ref[...] = jnp.maximum(y + be_ref[0], 0.0)


def _gmm(block_expert, xs, We, be):
    ne, d, u = We.shape
    grid_spec = pltpu.PrefetchScalarGridSpec(
        num_scalar_prefetch=1,
        grid=(NBMAX,),
        in_specs=[
            pl.BlockSpec((BLK, d), lambda b, bexp: (b, 0)),
            pl.BlockSpec((1, d, u), lambda b, bexp: (bexp[b], 0, 0)),
            pl.BlockSpec((1, 1, u), lambda b, bexp: (bexp[b], 0, 0)),
        ],
        out_specs=pl.BlockSpec((BLK, u), lambda b, bexp: (b, 0)),
    )
    return pl.pallas_call(
        _gmm_body,
        grid_spec=grid_spec,
        out_shape=jax.ShapeDtypeStruct((CAP, u), jnp.float32),
        compiler_params=pltpu.CompilerParams(dimension_semantics=("arbitrary",)),
    )(block_expert, xs, We, be.reshape(ne, 1, u))


def _combine_body(
    y_hbm,
    s0_hbm,
    s1_hbm,
    w0_hbm,
    w1_hbm,
    out_hbm,
    r0_v,
    r1_v,
    o_v,
    i0_v,
    i1_v,
    w0_v,
    w1_v,
    sem0,
    sem1,
):
    wid = lax.axis_index("s") * NC + lax.axis_index("c")
    tpc = 16
    d = 1024

    def body(ch, _):
        base = wid * TPW + ch * tpc
        pltpu.sync_copy(s0_hbm.at[pl.ds(base, tpc)], i0_v)
        pltpu.sync_copy(s1_hbm.at[pl.ds(base, tpc)], i1_v)
        pltpu.sync_copy(w0_hbm.at[pl.ds(base, tpc)], w0_v)
        pltpu.sync_copy(w1_hbm.at[pl.ds(base, tpc)], w1_v)
        g0 = pltpu.async_copy(y_hbm.at[i0_v], r0_v, sem0)
        g1 = pltpu.async_copy(y_hbm.at[i1_v], r1_v, sem1)
        g0.wait()
        g1.wait()

        w0vec = w0_v[...]
        w1vec = w1_v[...]

        def tok(i, _):
            i_vec = lax.broadcast_in_dim(i, (16,), ())
            wa = w0vec.at[i_vec].get(mode="promise_in_bounds")
            wb = w1vec.at[i_vec].get(mode="promise_in_bounds")

            def vec(v, _):
                sl = pl.ds(v * 16, 16)
                o_v[i, sl] = wa * r0_v[i, sl] + wb * r1_v[i, sl]
                return 0

            lax.fori_loop(0, d // 16, vec, 0)
            return 0

        lax.fori_loop(0, tpc, tok, 0)
        pltpu.sync_copy(o_v, out_hbm.at[pl.ds(base, tpc)])
        return 0

    lax.fori_loop(0, TPW // tpc, body, 0)


def _combine(y, s0, s1, w0, w1):
    u = y.shape[1]
    mesh = plsc.VectorSubcoreMesh(
        core_axis_name="c", subcore_axis_name="s", num_cores=NC, num_subcores=NS
    )
    f = functools.partial(
        pl.kernel,
        out_type=jax.ShapeDtypeStruct((NT, u), jnp.float32),
        mesh=mesh,
        scratch_types=[
            pltpu.VMEM((16, u), jnp.float32),
            pltpu.VMEM((16, u), jnp.float32),
            pltpu.VMEM((16, u), jnp.float32),
            pltpu.VMEM((16,), jnp.int32),
            pltpu.VMEM((16,), jnp.int32),
            pltpu.VMEM((16,), jnp.float32),
            pltpu.VMEM((16,), jnp.float32),
            pltpu.SemaphoreType.DMA,
            pltpu.SemaphoreType.DMA,
        ],
    )(_combine_body)
    return f(y, s0, s1, w0, w1)


def kernel(inputs, Wr, br, We, be):
    probs = _router(inputs, Wr, br)
    slots, w, block_expert = _routing(probs)
    slots = slots[:, 0]
    s0, s1 = slots[:NT], slots[NT:]
    xs = _dispatch(inputs, s0, s1)
    y = _gmm(block_expert[:, 0], xs, We, be)
    return _combine(y, s0, s1, w[:, 0], w[:, 1])
